# Initial kernel scaffold; baseline (speedup 1.0000x reference)
import jax
import jax.numpy as jnp
from jax.experimental import pallas as pl


def kernel(x, edge_index, W1, b1, W2, b2):
    raise NotImplementedError("write your pallas kernel here")


# speedup vs baseline: 28.9226x; 28.9226x over previous
import functools
import jax
import jax.numpy as jnp
from jax import lax
from jax.experimental import pallas as pl
from jax.experimental.pallas import tpu as pltpu
from jax.experimental.pallas import tpu_sc as plsc

N = 10000
NP = 10240
E = 320000
EP = 327680
F = 128
H = 16
NC = 2
NS = 16
NW = NC * NS
EB = 128
ROWS = EP // EB
RW = ROWS // NW
NPS = NP // NS


def _sc_body(gather, src_hbm, dst_hbm, table_hbm, fill_hbm, out_hbm,
             src_idx, dst_idx, rows, nbuf, acc):
    c = lax.axis_index("c")
    s = lax.axis_index("s")
    w = c * NS + s

    pltpu.sync_copy(fill_hbm.at[pl.ds(0, NPS)], nbuf)
    pltpu.sync_copy(nbuf, acc.at[pl.ds(s * NPS, NPS)])
    if not gather:
        pltpu.sync_copy(fill_hbm.at[pl.ds(NPS, EB)], rows)
    plsc.subcore_barrier()

    pltpu.sync_copy(dst_hbm.at[pl.ds(w * RW, RW)], dst_idx)
    if gather:
        pltpu.sync_copy(src_hbm.at[pl.ds(w * RW, RW)], src_idx)

    def step(j, _):
        if gather:
            pltpu.sync_copy(table_hbm.at[src_idx.at[j]], rows)
        pltpu.sync_copy(rows, acc.at[dst_idx.at[j]], add=True)
        return _

    lax.fori_loop(0, RW, step, None)
    plsc.subcore_barrier()

    pltpu.sync_copy(acc.at[pl.ds(s * NPS, NPS)], nbuf)
    pltpu.sync_copy(nbuf, out_hbm.at[pl.ds((c * NP) + s * NPS, NPS)])


def _make_sc_pass(gather):
    mesh = plsc.VectorSubcoreMesh(core_axis_name="c", subcore_axis_name="s")
    scratch = [
        pltpu.VMEM((RW, EB), jnp.int32),
        pltpu.VMEM((RW, EB), jnp.int32),
        pltpu.VMEM((EB, H), jnp.float32),
        pltpu.VMEM((NPS, H), jnp.float32),
        pltpu.VMEM_SHARED((NP, H), jnp.float32),
    ]
    return pl.kernel(
        functools.partial(_sc_body, gather),
        out_type=jax.ShapeDtypeStruct((NC * NP, H), jnp.float32),
        mesh=mesh,
        scratch_types=scratch,
        compiler_params=pltpu.CompilerParams(use_tc_tiling_on_sc=False),
        name="gcn_scatter" if gather else "gcn_degree",
    )


RB = 1024
GRID = NP // RB


def _tc1_body(x, w1, h0, h1, u_out, dinv_out):
    deg = h0[:, 0:1] + h1[:, 0:1] + 1.0
    dinv = lax.rsqrt(deg)
    h = jnp.dot(x[...], w1[...], preferred_element_type=jnp.float32)
    u_out[...] = h * dinv
    dinv_out[...] = dinv


def _tc2_body(q0, q1, u, dinv, b1, w2, g_out):
    s = q0[...] + q1[...] + u[...]
    l1 = jnp.maximum(dinv[...] * s + b1[...], 0.0)
    g = jnp.dot(l1, w2[...], preferred_element_type=jnp.float32)
    g_out[...] = g * dinv[...]


def _tc3_body(r0, r1, g, dinv, b2, o_out):
    o_out[...] = dinv[...] * (r0[...] + r1[...] + g[...]) + b2[...]


def _part_specs():
    return [
        pl.BlockSpec((RB, H), _row0),
        pl.BlockSpec((RB, H), _row1),
    ]


_row0 = lambda i: (i, 0)
_row1 = lambda i: (i + GRID, 0)
_full = lambda i: (0, 0)


def _tc1(x, w1, hist):
    return pl.pallas_call(
        _tc1_body,
        grid=(GRID,),
        in_specs=[
            pl.BlockSpec((RB, F), _row0),
            pl.BlockSpec((F, H), _full),
            *_part_specs(),
        ],
        out_specs=[
            pl.BlockSpec((RB, H), _row0),
            pl.BlockSpec((RB, 1), _row0),
        ],
        out_shape=[
            jax.ShapeDtypeStruct((NP, H), jnp.float32),
            jax.ShapeDtypeStruct((NP, 1), jnp.float32),
        ],
    )(x, w1, hist, hist)


def _tc2(q, u, dinv, b1, w2):
    return pl.pallas_call(
        _tc2_body,
        grid=(GRID,),
        in_specs=[
            *_part_specs(),
            pl.BlockSpec((RB, H), _row0),
            pl.BlockSpec((RB, 1), _row0),
            pl.BlockSpec((1, H), _full),
            pl.BlockSpec((H, H), _full),
        ],
        out_specs=pl.BlockSpec((RB, H), _row0),
        out_shape=jax.ShapeDtypeStruct((NP, H), jnp.float32),
    )(q, q, u, dinv, b1, w2)


def _tc3(r, g, dinv, b2):
    return pl.pallas_call(
        _tc3_body,
        grid=(GRID,),
        in_specs=[
            *_part_specs(),
            pl.BlockSpec((RB, H), _row0),
            pl.BlockSpec((RB, 1), _row0),
            pl.BlockSpec((1, H), _full),
        ],
        out_specs=pl.BlockSpec((RB, H), _row0),
        out_shape=jax.ShapeDtypeStruct((NP, H), jnp.float32),
    )(r, r, g, dinv, b2)


_hist_pass = _make_sc_pass(gather=False)
_msg_pass = _make_sc_pass(gather=True)


@jax.jit
def kernel(x, edge_index, W1, b1, W2, b2):
    pad = jnp.full((EP - E,), N, jnp.int32)
    src = jnp.concatenate([edge_index[0], pad]).reshape(ROWS, EB)
    dst = jnp.concatenate([edge_index[1], pad]).reshape(ROWS, EB)
    fill = jnp.concatenate(
        [jnp.zeros((NPS, H), jnp.float32), jnp.ones((EB, H), jnp.float32)])
    dummy_table = jnp.zeros((NP, H), jnp.float32)

    w2p = jnp.zeros((H, H), jnp.float32).at[:, :W2.shape[1]].set(W2)
    b1r = b1.reshape(1, H)
    b2p = jnp.zeros((1, H), jnp.float32).at[0, :b2.shape[0]].set(b2)

    hist = _hist_pass(src, dst, dummy_table, fill)
    u, dinv = _tc1(x, W1, hist)
    q = _msg_pass(src, dst, u, fill)
    g = _tc2(q, u, dinv, b1r, w2p)
    r = _msg_pass(src, dst, g, fill)
    out = _tc3(r, g, dinv, b2p)
    return out[:N, :b2.shape[0]]


# speedup vs baseline: 37.3489x; 1.2913x over previous
import functools
import jax
import jax.numpy as jnp
from jax import lax
from jax.experimental import pallas as pl
from jax.experimental.pallas import tpu as pltpu
from jax.experimental.pallas import tpu_sc as plsc

N = 10000
NP = 10240
E = 320000
EP = 327680
F = 128
H = 16
NC = 2
NS = 16
NW = NC * NS
EB = 128
ROWS = EP // EB
RW = ROWS // NW
NPS = NP // NS


NB = 4
NG = RW // (2 * NB)
NBD = 8
NGD = RW // NBD


def _sc_body(gather, src_hbm, dst_hbm, table_hbm, fill_hbm, out_hbm,
             src_idx, dst_idx, rows, nbuf, acc, gsem0, gsem1, ssem0, ssem1):
    c = lax.axis_index("c")
    s = lax.axis_index("s")
    w = c * NS + s

    pltpu.sync_copy(fill_hbm.at[pl.ds(0, NPS)], nbuf)
    pltpu.sync_copy(nbuf, acc.at[pl.ds(s * NPS, NPS)])
    plsc.subcore_barrier()

    pltpu.sync_copy(dst_hbm.at[pl.ds(w * RW, RW)], dst_idx)
    if gather:
        pltpu.sync_copy(src_hbm.at[pl.ds(w * RW, RW)], src_idx)

    def drain(sem, buf, n):
        for _ in range(n):
            pltpu.make_async_copy(fill_hbm.at[pl.ds(0, EB)], buf, sem).wait()

    if gather:
        gsems = (gsem0, gsem1)
        ssems = (ssem0, ssem1)

        def step(k, _):
            for st in (0, 1):
                g = 2 * k + st

                @pl.when(k > 0)
                def _():
                    drain(ssems[st], rows.at[st, 0], NB)

                for b in range(NB):
                    pltpu.async_copy(
                        table_hbm.at[src_idx.at[g * NB + b]],
                        rows.at[st, b], gsems[st])
            for st in (0, 1):
                g = 2 * k + st
                drain(gsems[st], rows.at[st, 0], NB)
                for b in range(NB):
                    pltpu.async_copy(
                        rows.at[st, b],
                        acc.at[dst_idx.at[g * NB + b]], ssems[st], add=True)
            return _

        lax.fori_loop(0, NG, step, None)
        drain(ssem0, rows.at[0, 0], NB)
        drain(ssem1, rows.at[1, 0], NB)
    else:
        pltpu.sync_copy(fill_hbm.at[pl.ds(NPS, EB)], nbuf.at[pl.ds(0, EB)])
        ones = nbuf.at[pl.ds(0, EB)]

        def stepd(k, _):
            @pl.when(k > 0)
            def _():
                drain(ssem0, ones, NBD)

            for b in range(NBD):
                pltpu.async_copy(
                    ones, acc.at[dst_idx.at[k * NBD + b]], ssem0, add=True)
            return _

        lax.fori_loop(0, NGD, stepd, None)
        drain(ssem0, ones, NBD)

    plsc.subcore_barrier()

    pltpu.sync_copy(acc.at[pl.ds(s * NPS, NPS)], nbuf)
    pltpu.sync_copy(nbuf, out_hbm.at[pl.ds((c * NP) + s * NPS, NPS)])


def _make_sc_pass(gather):
    mesh = plsc.VectorSubcoreMesh(core_axis_name="c", subcore_axis_name="s")
    scratch = [
        pltpu.VMEM((RW, EB), jnp.int32),
        pltpu.VMEM((RW, EB), jnp.int32),
        pltpu.VMEM((2, NB, EB, H), jnp.float32),
        pltpu.VMEM((NPS, H), jnp.float32),
        pltpu.VMEM_SHARED((NP, H), jnp.float32),
        pltpu.SemaphoreType.DMA,
        pltpu.SemaphoreType.DMA,
        pltpu.SemaphoreType.DMA,
        pltpu.SemaphoreType.DMA,
    ]
    return pl.kernel(
        functools.partial(_sc_body, gather),
        out_type=jax.ShapeDtypeStruct((NC * NP, H), jnp.float32),
        mesh=mesh,
        scratch_types=scratch,
        compiler_params=pltpu.CompilerParams(use_tc_tiling_on_sc=False),
        name="gcn_scatter" if gather else "gcn_degree",
    )


RB = 1024
GRID = NP // RB


def _tc1_body(x, w1, h0, h1, u_out, dinv_out):
    deg = h0[:, 0:1] + h1[:, 0:1] + 1.0
    dinv = lax.rsqrt(deg)
    h = jnp.dot(x[...], w1[...], preferred_element_type=jnp.float32)
    u_out[...] = h * dinv
    dinv_out[...] = dinv


def _tc2_body(q0, q1, u, dinv, b1, w2, g_out):
    s = q0[...] + q1[...] + u[...]
    l1 = jnp.maximum(dinv[...] * s + b1[...], 0.0)
    g = jnp.dot(l1, w2[...], preferred_element_type=jnp.float32)
    g_out[...] = g * dinv[...]


def _tc3_body(r0, r1, g, dinv, b2, o_out):
    o_out[...] = dinv[...] * (r0[...] + r1[...] + g[...]) + b2[...]


def _part_specs():
    return [
        pl.BlockSpec((RB, H), _row0),
        pl.BlockSpec((RB, H), _row1),
    ]


_row0 = lambda i: (i, 0)
_row1 = lambda i: (i + GRID, 0)
_full = lambda i: (0, 0)


def _tc1(x, w1, hist):
    return pl.pallas_call(
        _tc1_body,
        grid=(GRID,),
        in_specs=[
            pl.BlockSpec((RB, F), _row0),
            pl.BlockSpec((F, H), _full),
            *_part_specs(),
        ],
        out_specs=[
            pl.BlockSpec((RB, H), _row0),
            pl.BlockSpec((RB, 1), _row0),
        ],
        out_shape=[
            jax.ShapeDtypeStruct((NP, H), jnp.float32),
            jax.ShapeDtypeStruct((NP, 1), jnp.float32),
        ],
    )(x, w1, hist, hist)


def _tc2(q, u, dinv, b1, w2):
    return pl.pallas_call(
        _tc2_body,
        grid=(GRID,),
        in_specs=[
            *_part_specs(),
            pl.BlockSpec((RB, H), _row0),
            pl.BlockSpec((RB, 1), _row0),
            pl.BlockSpec((1, H), _full),
            pl.BlockSpec((H, H), _full),
        ],
        out_specs=pl.BlockSpec((RB, H), _row0),
        out_shape=jax.ShapeDtypeStruct((NP, H), jnp.float32),
    )(q, q, u, dinv, b1, w2)


def _tc3(r, g, dinv, b2):
    return pl.pallas_call(
        _tc3_body,
        grid=(GRID,),
        in_specs=[
            *_part_specs(),
            pl.BlockSpec((RB, H), _row0),
            pl.BlockSpec((RB, 1), _row0),
            pl.BlockSpec((1, H), _full),
        ],
        out_specs=pl.BlockSpec((RB, H), _row0),
        out_shape=jax.ShapeDtypeStruct((NP, H), jnp.float32),
    )(r, r, g, dinv, b2)


_hist_pass = _make_sc_pass(gather=False)
_msg_pass = _make_sc_pass(gather=True)


@jax.jit
def kernel(x, edge_index, W1, b1, W2, b2):
    pad = jnp.full((EP - E,), N, jnp.int32)
    src = jnp.concatenate([edge_index[0], pad]).reshape(ROWS, EB)
    dst = jnp.concatenate([edge_index[1], pad]).reshape(ROWS, EB)
    fill = jnp.concatenate(
        [jnp.zeros((NPS, H), jnp.float32), jnp.ones((EB, H), jnp.float32)])
    dummy_table = jnp.zeros((NP, H), jnp.float32)

    w2p = jnp.zeros((H, H), jnp.float32).at[:, :W2.shape[1]].set(W2)
    b1r = b1.reshape(1, H)
    b2p = jnp.zeros((1, H), jnp.float32).at[0, :b2.shape[0]].set(b2)

    hist = _hist_pass(src, dst, dummy_table, fill)
    u, dinv = _tc1(x, W1, hist)
    q = _msg_pass(src, dst, u, fill)
    g = _tc2(q, u, dinv, b1r, w2p)
    r = _msg_pass(src, dst, g, fill)
    out = _tc3(r, g, dinv, b2p)
    return out[:N, :b2.shape[0]]
